# baseline (device time: 16370 ns/iter reference)
import jax
import jax.numpy as jnp
from jax import lax
from jax.experimental import pallas as pl
from jax.experimental.pallas import tpu as pltpu

CHUNK_ROWS = (128, 128, 128, 96, 16, 16)
C = len(CHUNK_ROWS)
CHUNK_OFF = tuple(sum(CHUNK_ROWS[:i]) for i in range(C))


def kernel(x):
    m, n = x.shape
    H = m // 2
    assert sum(CHUNK_ROWS) == H

    def body(
        x_hbm,
        out_hbm,
        xv,
        obuf,
        ybuf,
        in_sems,
        osem,
        ldsems,
        ysend,
        yrecv,
        xsend,
        xrecv,
    ):
        my_x = lax.axis_index("x")
        my_y = lax.axis_index("y")
        ynbr = (my_x, 1 - my_y)
        xnbr = (1 - my_x, my_y)

        my_row0 = my_y * m
        wire0 = my_x * H
        yland0 = (1 - my_y) * m + my_x * H

        in_a = pltpu.make_async_copy(
            x_hbm.at[pl.ds(wire0, H)], xv.at[pl.ds(0, H)], in_sems.at[0]
        )
        in_a.start()
        oth0 = (1 - my_x) * H
        in_b = pltpu.make_async_copy(
            x_hbm.at[pl.ds(oth0, H)], xv.at[pl.ds(H, H)], in_sems.at[1]
        )
        in_b.start()

        barrier = pltpu.get_barrier_semaphore()
        for nbr in (ynbr, xnbr):
            pl.semaphore_signal(
                barrier, inc=1, device_id=nbr, device_id_type=pl.DeviceIdType.MESH
            )

        in_a.wait()
        obuf[pl.ds(wire0, CHUNK_ROWS[0]), :] = xv[
            pl.ds(0, CHUNK_ROWS[0]), :
        ].astype(jnp.bfloat16)

        pl.semaphore_wait(barrier, 2)

        yrdmas = []
        for c in range(C):
            rows = CHUNK_ROWS[c]
            off = CHUNK_OFF[c]
            if c > 0:
                obuf[pl.ds(wire0 + off, rows), :] = xv[pl.ds(off, rows), :].astype(
                    jnp.bfloat16
                )
            r = pltpu.make_async_remote_copy(
                src_ref=obuf.at[pl.ds(wire0 + off, rows)],
                dst_ref=ybuf.at[pl.ds(off, rows)],
                send_sem=ysend.at[c],
                recv_sem=yrecv.at[c],
                device_id=ynbr,
                device_id_type=pl.DeviceIdType.MESH,
            )
            r.start()
            yrdmas.append(r)

        in_b.wait()
        obuf[pl.ds(oth0, H), :] = xv[pl.ds(H, H), :].astype(jnp.bfloat16)
        own = pltpu.make_async_copy(obuf, out_hbm.at[pl.ds(my_row0, m)], osem)
        own.start()

        xrdmas = []
        lds = []
        for c in range(C):
            rows = CHUNK_ROWS[c]
            off = CHUNK_OFF[c]
            vsl = pl.ds(off, rows)
            gsl = pl.ds(yland0 + off, rows)
            yrdmas[c].wait_recv()
            r = pltpu.make_async_remote_copy(
                src_ref=ybuf.at[vsl],
                dst_ref=out_hbm.at[gsl],
                send_sem=xsend.at[c],
                recv_sem=xrecv.at[c],
                device_id=xnbr,
                device_id_type=pl.DeviceIdType.MESH,
            )
            r.start()
            xrdmas.append(r)
            ld = pltpu.make_async_copy(ybuf.at[vsl], out_hbm.at[gsl], ldsems.at[c])
            ld.start()
            lds.append(ld)

        for c in range(C):
            xrdmas[c].wait_recv()
        own.wait()
        for c in range(C):
            lds[c].wait()
            yrdmas[c].wait_send()
            xrdmas[c].wait_send()

    return pl.pallas_call(
        body,
        out_shape=jax.ShapeDtypeStruct((2 * m, n), jnp.bfloat16),
        in_specs=[pl.BlockSpec(memory_space=pl.ANY)],
        out_specs=pl.BlockSpec(memory_space=pl.ANY),
        scratch_shapes=[
            pltpu.VMEM((m, n), jnp.float32),
            pltpu.VMEM((m, n), jnp.bfloat16),
            pltpu.VMEM((H, n), jnp.bfloat16),
            pltpu.SemaphoreType.DMA((2,)),
            pltpu.SemaphoreType.DMA,
            pltpu.SemaphoreType.DMA((C,)),
            pltpu.SemaphoreType.DMA((C,)),
            pltpu.SemaphoreType.DMA((C,)),
            pltpu.SemaphoreType.DMA((C,)),
            pltpu.SemaphoreType.DMA((C,)),
        ],
        compiler_params=pltpu.CompilerParams(collective_id=0),
    )(x)


# device time: 15540 ns/iter; 1.0534x vs baseline; 1.0534x over previous
import jax
import jax.numpy as jnp
from jax import lax
from jax.experimental import pallas as pl
from jax.experimental.pallas import tpu as pltpu

C = 8


def kernel(x):
    m, n = x.shape
    H = m // 2
    R = H // C

    def body(x_ref, out_ref, ysend, yrecv, xsend, xrecv):
        my_x = lax.axis_index("x")
        my_y = lax.axis_index("y")
        ynbr = (my_x, 1 - my_y)
        xnbr = (1 - my_x, my_y)

        my_row0 = my_y * m
        wire0 = my_row0 + my_x * H
        yland0 = (1 - my_y) * m + my_x * H
        xland0 = (1 - my_y) * m + (1 - my_x) * H

        barrier = pltpu.get_barrier_semaphore()
        for nbr in (ynbr, xnbr):
            pl.semaphore_signal(
                barrier, inc=1, device_id=nbr, device_id_type=pl.DeviceIdType.MESH
            )
        pl.semaphore_wait(barrier, 2)

        yrdmas = []
        for c in range(C):
            src_sl = pl.ds(wire0 + c * R, R)
            out_ref[src_sl, :] = x_ref[pl.ds(my_x * H + c * R, R), :].astype(
                jnp.bfloat16
            )
            r = pltpu.make_async_remote_copy(
                src_ref=out_ref.at[src_sl],
                dst_ref=out_ref.at[src_sl],
                send_sem=ysend.at[c],
                recv_sem=yrecv.at[c],
                device_id=ynbr,
                device_id_type=pl.DeviceIdType.MESH,
            )
            r.start()
            yrdmas.append(r)

        oth = 1 - my_x
        out_ref[pl.ds(my_row0 + oth * H, H), :] = x_ref[pl.ds(oth * H, H), :].astype(
            jnp.bfloat16
        )

        xrdmas = []
        for c in range(C):
            sl = pl.ds(yland0 + c * R, R)
            yrdmas[c].wait_recv()
            r = pltpu.make_async_remote_copy(
                src_ref=out_ref.at[sl],
                dst_ref=out_ref.at[sl],
                send_sem=xsend.at[c],
                recv_sem=xrecv.at[c],
                device_id=xnbr,
                device_id_type=pl.DeviceIdType.MESH,
            )
            r.start()
            xrdmas.append(r)

        for c in range(C):
            xrdmas[c].wait_recv()
        for c in range(C):
            yrdmas[c].wait_send()
            xrdmas[c].wait_send()
        del xland0

    return pl.pallas_call(
        body,
        out_shape=jax.ShapeDtypeStruct((2 * m, n), jnp.bfloat16),
        in_specs=[pl.BlockSpec(memory_space=pltpu.VMEM)],
        out_specs=pl.BlockSpec(memory_space=pltpu.VMEM),
        scratch_shapes=[
            pltpu.SemaphoreType.DMA((C,)),
            pltpu.SemaphoreType.DMA((C,)),
            pltpu.SemaphoreType.DMA((C,)),
            pltpu.SemaphoreType.DMA((C,)),
        ],
        compiler_params=pltpu.CompilerParams(collective_id=0),
    )(x)


# device time: 14614 ns/iter; 1.1202x vs baseline; 1.0634x over previous
import jax
import jax.numpy as jnp
from jax import lax
from jax.experimental import pallas as pl
from jax.experimental.pallas import tpu as pltpu

C = 16


def kernel(x):
    m, n = x.shape
    H = m // 2
    R = H // C

    def body(x_hbm, out_ref, xv, in_sems, ysend, yrecv, xsend, xrecv):
        my_x = lax.axis_index("x")
        my_y = lax.axis_index("y")
        ynbr = (my_x, 1 - my_y)
        xnbr = (1 - my_x, my_y)

        my_row0 = my_y * m
        wire0 = my_row0 + my_x * H
        yland0 = (1 - my_y) * m + my_x * H
        src0 = my_x * H

        in_0 = pltpu.make_async_copy(
            x_hbm.at[pl.ds(src0, R)], xv.at[pl.ds(src0, R)], in_sems.at[0]
        )
        in_0.start()
        in_a = pltpu.make_async_copy(
            x_hbm.at[pl.ds(src0 + R, H - R)],
            xv.at[pl.ds(src0 + R, H - R)],
            in_sems.at[1],
        )
        in_a.start()
        oth0 = (1 - my_x) * H
        in_b = pltpu.make_async_copy(
            x_hbm.at[pl.ds(oth0, H)], xv.at[pl.ds(oth0, H)], in_sems.at[2]
        )
        in_b.start()

        barrier = pltpu.get_barrier_semaphore()
        for nbr in (ynbr, xnbr):
            pl.semaphore_signal(
                barrier, inc=1, device_id=nbr, device_id_type=pl.DeviceIdType.MESH
            )
        pl.semaphore_wait(barrier, 2)

        in_0.wait()
        yrdmas = []
        for c in range(C):
            if c == 1:
                in_a.wait()
            sl = pl.ds(wire0 + c * R, R)
            out_ref[sl, :] = xv[pl.ds(src0 + c * R, R), :].astype(jnp.bfloat16)
            r = pltpu.make_async_remote_copy(
                src_ref=out_ref.at[sl],
                dst_ref=out_ref.at[sl],
                send_sem=ysend.at[c],
                recv_sem=yrecv.at[c],
                device_id=ynbr,
                device_id_type=pl.DeviceIdType.MESH,
            )
            r.start()
            yrdmas.append(r)

        in_b.wait()
        out_ref[pl.ds(my_row0 + oth0, H), :] = xv[pl.ds(oth0, H), :].astype(
            jnp.bfloat16
        )

        xrdmas = []
        for c in range(C):
            sl = pl.ds(yland0 + c * R, R)
            yrdmas[c].wait_recv()
            r = pltpu.make_async_remote_copy(
                src_ref=out_ref.at[sl],
                dst_ref=out_ref.at[sl],
                send_sem=xsend.at[c],
                recv_sem=xrecv.at[c],
                device_id=xnbr,
                device_id_type=pl.DeviceIdType.MESH,
            )
            r.start()
            xrdmas.append(r)

        for c in range(C):
            xrdmas[c].wait_recv()
        for c in range(C):
            yrdmas[c].wait_send()
            xrdmas[c].wait_send()

    return pl.pallas_call(
        body,
        out_shape=jax.ShapeDtypeStruct((2 * m, n), jnp.bfloat16),
        in_specs=[pl.BlockSpec(memory_space=pltpu.MemorySpace.HBM)],
        out_specs=pl.BlockSpec(memory_space=pltpu.VMEM),
        scratch_shapes=[
            pltpu.VMEM((m, n), jnp.float32),
            pltpu.SemaphoreType.DMA((3,)),
            pltpu.SemaphoreType.DMA((C,)),
            pltpu.SemaphoreType.DMA((C,)),
            pltpu.SemaphoreType.DMA((C,)),
            pltpu.SemaphoreType.DMA((C,)),
        ],
        compiler_params=pltpu.CompilerParams(collective_id=0),
    )(pltpu.with_memory_space_constraint(x, pltpu.MemorySpace.HBM))
